# initial kernel scaffold (unmeasured)
import jax
import jax.numpy as jnp
from jax import lax
from jax.experimental import pallas as pl
from jax.experimental.pallas import tpu as pltpu

N_DEV = 8
N_LANES = 1


def kernel(x, w_mat):
    x = x.astype(jnp.bfloat16)
    w_mat = w_mat.astype(jnp.bfloat16)

    M, _ = x.shape
    _, N = w_mat.shape
    Mc = M // N_DEV
    W = N // N_LANES
    DIRS = [1, -1][:N_LANES]
    NSTEP = 2 * (N_DEV - 1)

    def body(x_ref, w_ref, out_ref, *scr):
        comms = scr[:N_LANES]
        send_sems, recv_sems, out_sems = scr[N_LANES : N_LANES + 3]
        credits = scr[N_LANES + 3 :]

        my = lax.axis_index("i")
        left = (my - 1) % N_DEV
        right = (my + 1) % N_DEV
        tgt = [(my + d) % N_DEV for d in DIRS]
        src = [(my - d) % N_DEV for d in DIRS]

        barrier_sem = pltpu.get_barrier_semaphore()
        for nbr in (left, right):
            pl.semaphore_signal(
                barrier_sem, inc=1,
                device_id=(nbr,), device_id_type=pl.DeviceIdType.MESH,
            )
        pl.semaphore_wait(barrier_sem, 2)

        def partial_chunk(c, li):
            xs = x_ref[pl.ds(c * Mc, Mc), :]
            ws = w_ref[:, li * W : (li + 1) * W]
            return jnp.dot(xs, ws, preferred_element_type=jnp.float32)

        for li in range(N_LANES):
            comms[li][0, :, :] = partial_chunk(my, li).astype(jnp.bfloat16)

        for g in range(NSTEP):
            ss, rs = g % 2, (g + 1) % 2
            rdmas = []
            for li in range(N_LANES):
                if g >= 1:
                    pl.semaphore_wait(credits[li], 1)
                rdma = pltpu.make_async_remote_copy(
                    src_ref=comms[li].at[ss],
                    dst_ref=comms[li].at[rs],
                    send_sem=send_sems.at[li, ss],
                    recv_sem=recv_sems.at[li, rs],
                    device_id=(tgt[li],),
                    device_id_type=pl.DeviceIdType.MESH,
                )
                rdma.start()
                rdmas.append(rdma)

            ps = None
            if g < N_DEV - 1:
                ps = [
                    partial_chunk((my - DIRS[li] * (g + 1)) % N_DEV, li)
                    for li in range(N_LANES)
                ]

            for li in range(N_LANES):
                rdmas[li].wait()
                if g < N_DEV - 1:
                    acc = comms[li][rs, :, :].astype(jnp.float32) + ps[li]
                    if g == N_DEV - 2:
                        acc = jnp.maximum(acc, 0.0)
                    comms[li][rs, :, :] = acc.astype(jnp.bfloat16)
                    if g == N_DEV - 2:
                        own = (my + DIRS[li]) % N_DEV
                        cp = pltpu.make_async_copy(
                            comms[li].at[rs],
                            out_ref.at[pl.ds(own * Mc, Mc), pl.ds(li * W, W)],
                            out_sems.at[li],
                        )
                        cp.start()
                        cp.wait()
                else:
                    t = g - (N_DEV - 1)
                    idx = (my - DIRS[li] * t) % N_DEV
                    cp = pltpu.make_async_copy(
                        comms[li].at[rs],
                        out_ref.at[pl.ds(idx * Mc, Mc), pl.ds(li * W, W)],
                        out_sems.at[li],
                    )
                    cp.start()
                    cp.wait()
                if g < NSTEP - 1:
                    pl.semaphore_signal(
                        credits[li], inc=1,
                        device_id=(src[li],),
                        device_id_type=pl.DeviceIdType.MESH,
                    )

    scratch = (
        [pltpu.VMEM((2, Mc, W), jnp.bfloat16) for _ in range(N_LANES)]
        + [
            pltpu.SemaphoreType.DMA((N_LANES, 2)),
            pltpu.SemaphoreType.DMA((N_LANES, 2)),
            pltpu.SemaphoreType.DMA((N_LANES,)),
        ]
        + [pltpu.SemaphoreType.REGULAR for _ in range(N_LANES)]
    )

    return pl.pallas_call(
        body,
        out_shape=jax.ShapeDtypeStruct((M, N), jnp.bfloat16),
        in_specs=[
            pl.BlockSpec(memory_space=pltpu.VMEM),
            pl.BlockSpec(memory_space=pltpu.VMEM),
        ],
        out_specs=pl.BlockSpec(memory_space=pltpu.MemorySpace.ANY),
        scratch_shapes=scratch,
        compiler_params=pltpu.CompilerParams(
            collective_id=0,
            vmem_limit_bytes=128 * 1024 * 1024,
        ),
    )(x, w_mat)


# baseline (device time: 1410896 ns/iter reference)
import jax
import jax.numpy as jnp
from jax import lax
from jax.experimental import pallas as pl
from jax.experimental.pallas import tpu as pltpu

N_DEV = 8
N_LANES = 1


def kernel(x, w_mat):
    x = x.astype(jnp.bfloat16)
    w_mat = w_mat.astype(jnp.bfloat16)

    M, _ = x.shape
    _, N = w_mat.shape
    Mc = M // N_DEV
    W = N // N_LANES
    DIRS = [1, -1][:N_LANES]
    NSTEP = 2 * (N_DEV - 1)

    def body(x_ref, w_ref, out_ref, *scr):
        comms = scr[:N_LANES]
        send_sems, recv_sems, out_sems = scr[N_LANES : N_LANES + 3]
        credits = scr[N_LANES + 3 :]

        my = lax.axis_index("i")
        left = (my - 1) % N_DEV
        right = (my + 1) % N_DEV
        tgt = [(my + d) % N_DEV for d in DIRS]
        src = [(my - d) % N_DEV for d in DIRS]

        barrier_sem = pltpu.get_barrier_semaphore()
        for nbr in (left, right):
            pl.semaphore_signal(
                barrier_sem, inc=1,
                device_id=(nbr,), device_id_type=pl.DeviceIdType.MESH,
            )
        pl.semaphore_wait(barrier_sem, 2)

        def partial_chunk(c, li):
            xs = x_ref[pl.ds(c * Mc, Mc), :]
            ws = w_ref[:, li * W : (li + 1) * W]
            return jnp.dot(xs, ws, preferred_element_type=jnp.float32)

        for li in range(N_LANES):
            comms[li][0, :, :] = partial_chunk(my, li).astype(jnp.bfloat16)

        for g in range(NSTEP):
            ss, rs = g % 2, (g + 1) % 2
            rdmas = []
            for li in range(N_LANES):
                if g >= 1:
                    pl.semaphore_wait(credits[li], 1)
                rdma = pltpu.make_async_remote_copy(
                    src_ref=comms[li].at[ss],
                    dst_ref=comms[li].at[rs],
                    send_sem=send_sems.at[li, ss],
                    recv_sem=recv_sems.at[li, rs],
                    device_id=(tgt[li],),
                    device_id_type=pl.DeviceIdType.MESH,
                )
                rdma.start()
                rdmas.append(rdma)

            ps = None
            if g < N_DEV - 1:
                ps = [
                    partial_chunk((my - DIRS[li] * (g + 1)) % N_DEV, li)
                    for li in range(N_LANES)
                ]

            for li in range(N_LANES):
                rdmas[li].wait()
                if g < N_DEV - 1:
                    acc = comms[li][rs, :, :].astype(jnp.float32) + ps[li]
                    if g == N_DEV - 2:
                        acc = jnp.maximum(acc, 0.0)
                    comms[li][rs, :, :] = acc.astype(jnp.bfloat16)
                    if g == N_DEV - 2:
                        own = (my + DIRS[li]) % N_DEV
                        cp = pltpu.make_async_copy(
                            comms[li].at[rs],
                            out_ref.at[pl.ds(own * Mc, Mc), pl.ds(li * W, W)],
                            out_sems.at[li],
                        )
                        cp.start()
                        cp.wait()
                else:
                    t = g - (N_DEV - 1)
                    idx = (my - DIRS[li] * t) % N_DEV
                    cp = pltpu.make_async_copy(
                        comms[li].at[rs],
                        out_ref.at[pl.ds(idx * Mc, Mc), pl.ds(li * W, W)],
                        out_sems.at[li],
                    )
                    cp.start()
                    cp.wait()
                if g < NSTEP - 1:
                    pl.semaphore_signal(
                        credits[li], inc=1,
                        device_id=(src[li],),
                        device_id_type=pl.DeviceIdType.MESH,
                    )

    scratch = (
        [pltpu.VMEM((2, Mc, W), jnp.bfloat16) for _ in range(N_LANES)]
        + [
            pltpu.SemaphoreType.DMA((N_LANES, 2)),
            pltpu.SemaphoreType.DMA((N_LANES, 2)),
            pltpu.SemaphoreType.DMA((N_LANES,)),
        ]
        + [pltpu.SemaphoreType.REGULAR for _ in range(N_LANES)]
    )

    return pl.pallas_call(
        body,
        out_shape=jax.ShapeDtypeStruct((M, N), jnp.bfloat16),
        in_specs=[
            pl.BlockSpec(memory_space=pltpu.VMEM),
            pl.BlockSpec(memory_space=pltpu.VMEM),
        ],
        out_specs=pl.BlockSpec(memory_space=pl.ANY),
        scratch_shapes=scratch,
        compiler_params=pltpu.CompilerParams(
            collective_id=0,
            vmem_limit_bytes=128 * 1024 * 1024,
        ),
    )(x, w_mat)


# device time: 773051 ns/iter; 1.8251x vs baseline; 1.8251x over previous
import jax
import jax.numpy as jnp
from jax import lax
from jax.experimental import pallas as pl
from jax.experimental.pallas import tpu as pltpu

N_DEV = 8
N_LANES = 2


def kernel(x, w_mat):
    x = x.astype(jnp.bfloat16)
    w_mat = w_mat.astype(jnp.bfloat16)

    M, _ = x.shape
    _, N = w_mat.shape
    Mc = M // N_DEV
    W = N // N_LANES
    DIRS = [1, -1][:N_LANES]
    NSTEP = 2 * (N_DEV - 1)

    def body(x_ref, w_ref, out_ref, *scr):
        comms = scr[:N_LANES]
        send_sems, recv_sems, out_sems = scr[N_LANES : N_LANES + 3]
        credits = scr[N_LANES + 3 :]

        my = lax.axis_index("i")
        left = (my - 1) % N_DEV
        right = (my + 1) % N_DEV
        tgt = [(my + d) % N_DEV for d in DIRS]
        src = [(my - d) % N_DEV for d in DIRS]

        barrier_sem = pltpu.get_barrier_semaphore()
        for nbr in (left, right):
            pl.semaphore_signal(
                barrier_sem, inc=1,
                device_id=(nbr,), device_id_type=pl.DeviceIdType.MESH,
            )
        pl.semaphore_wait(barrier_sem, 2)

        def partial_chunk(c, li):
            xs = x_ref[pl.ds(c * Mc, Mc), :]
            ws = w_ref[:, li * W : (li + 1) * W]
            return jnp.dot(xs, ws, preferred_element_type=jnp.float32)

        for li in range(N_LANES):
            comms[li][0, :, :] = partial_chunk(my, li).astype(jnp.bfloat16)

        for g in range(NSTEP):
            ss, rs = g % 2, (g + 1) % 2
            rdmas = []
            for li in range(N_LANES):
                if g >= 1:
                    pl.semaphore_wait(credits[li], 1)
                rdma = pltpu.make_async_remote_copy(
                    src_ref=comms[li].at[ss],
                    dst_ref=comms[li].at[rs],
                    send_sem=send_sems.at[li, ss],
                    recv_sem=recv_sems.at[li, rs],
                    device_id=(tgt[li],),
                    device_id_type=pl.DeviceIdType.MESH,
                )
                rdma.start()
                rdmas.append(rdma)

            ps = None
            if g < N_DEV - 1:
                ps = [
                    partial_chunk((my - DIRS[li] * (g + 1)) % N_DEV, li)
                    for li in range(N_LANES)
                ]

            for li in range(N_LANES):
                rdmas[li].wait()
                if g < N_DEV - 1:
                    acc = comms[li][rs, :, :].astype(jnp.float32) + ps[li]
                    if g == N_DEV - 2:
                        acc = jnp.maximum(acc, 0.0)
                    comms[li][rs, :, :] = acc.astype(jnp.bfloat16)
                    if g == N_DEV - 2:
                        own = (my + DIRS[li]) % N_DEV
                        cp = pltpu.make_async_copy(
                            comms[li].at[rs],
                            out_ref.at[pl.ds(own * Mc, Mc), pl.ds(li * W, W)],
                            out_sems.at[li],
                        )
                        cp.start()
                        cp.wait()
                else:
                    t = g - (N_DEV - 1)
                    idx = (my - DIRS[li] * t) % N_DEV
                    cp = pltpu.make_async_copy(
                        comms[li].at[rs],
                        out_ref.at[pl.ds(idx * Mc, Mc), pl.ds(li * W, W)],
                        out_sems.at[li],
                    )
                    cp.start()
                    cp.wait()
                if g < NSTEP - 1:
                    pl.semaphore_signal(
                        credits[li], inc=1,
                        device_id=(src[li],),
                        device_id_type=pl.DeviceIdType.MESH,
                    )

    scratch = (
        [pltpu.VMEM((2, Mc, W), jnp.bfloat16) for _ in range(N_LANES)]
        + [
            pltpu.SemaphoreType.DMA((N_LANES, 2)),
            pltpu.SemaphoreType.DMA((N_LANES, 2)),
            pltpu.SemaphoreType.DMA((N_LANES,)),
        ]
        + [pltpu.SemaphoreType.REGULAR for _ in range(N_LANES)]
    )

    return pl.pallas_call(
        body,
        out_shape=jax.ShapeDtypeStruct((M, N), jnp.bfloat16),
        in_specs=[
            pl.BlockSpec(memory_space=pltpu.VMEM),
            pl.BlockSpec(memory_space=pltpu.VMEM),
        ],
        out_specs=pl.BlockSpec(memory_space=pl.ANY),
        scratch_shapes=scratch,
        compiler_params=pltpu.CompilerParams(
            collective_id=0,
            vmem_limit_bytes=128 * 1024 * 1024,
        ),
    )(x, w_mat)


# device time: 730753 ns/iter; 1.9307x vs baseline; 1.0579x over previous
import jax
import jax.numpy as jnp
from jax import lax
from jax.experimental import pallas as pl
from jax.experimental.pallas import tpu as pltpu

N_DEV = 8
N_LANES = 4

PERM = (0, 1, 2, 3, 7, 6, 5, 4)


def kernel(x, w_mat):
    x = x.astype(jnp.bfloat16)
    w_mat = w_mat.astype(jnp.bfloat16)

    M, _ = x.shape
    _, N = w_mat.shape
    Mc = M // N_DEV
    W = N // N_LANES
    DIRS = [1 if li % 2 == 0 else -1 for li in range(N_LANES)]
    COL0 = [(li % 2) * (N // 2) + (li // 2) * W for li in range(N_LANES)]
    NSTEP = 2 * (N_DEV - 1)

    def body(x_ref, w_ref, out_ref, *scr):
        comms = scr[:N_LANES]
        send_sems, recv_sems, out_sems = scr[N_LANES : N_LANES + 3]
        credits = scr[N_LANES + 3 :]

        def perm(v):
            return jnp.where(v < 4, v, 11 - v)

        my_mesh = lax.axis_index("i")
        my = perm(my_mesh)
        tgt = [perm((my + d) % N_DEV) for d in DIRS]
        src = [perm((my - d) % N_DEV) for d in DIRS]

        barrier_sem = pltpu.get_barrier_semaphore()
        for nbr in (tgt[0], src[0]):
            pl.semaphore_signal(
                barrier_sem, inc=1,
                device_id=(nbr,), device_id_type=pl.DeviceIdType.MESH,
            )
        pl.semaphore_wait(barrier_sem, 2)

        def partial_chunk(c, li):
            xs = x_ref[pl.ds(c * Mc, Mc), :]
            ws = w_ref[:, COL0[li] : COL0[li] + W]
            return jnp.dot(
                xs, ws, preferred_element_type=jnp.float32
            ).astype(jnp.bfloat16)

        for li in range(N_LANES):
            comms[li][0, :, :] = partial_chunk(my, li)

        pending = [None] * N_LANES

        for g in range(NSTEP):
            ss, rs = g % 2, (g + 1) % 2
            rdmas = []
            for li in range(N_LANES):
                if g >= 1:
                    pl.semaphore_wait(credits[li], 1)
                rdma = pltpu.make_async_remote_copy(
                    src_ref=comms[li].at[ss],
                    dst_ref=comms[li].at[rs],
                    send_sem=send_sems.at[li, ss],
                    recv_sem=recv_sems.at[li, rs],
                    device_id=(tgt[li],),
                    device_id_type=pl.DeviceIdType.MESH,
                )
                rdma.start()
                rdmas.append(rdma)

            ps = None
            if g < N_DEV - 1:
                ps = [
                    partial_chunk((my - DIRS[li] * (g + 1)) % N_DEV, li)
                    for li in range(N_LANES)
                ]

            for li in range(N_LANES):
                rdmas[li].wait()
                if pending[li] is not None:
                    pending[li].wait()
                    pending[li] = None
                if g < N_DEV - 1:
                    acc = comms[li][rs, :, :] + ps[li]
                    if g == N_DEV - 2:
                        acc = jnp.maximum(acc, 0)
                    comms[li][rs, :, :] = acc
                    if g == N_DEV - 2:
                        own = (my + DIRS[li]) % N_DEV
                        cp = pltpu.make_async_copy(
                            comms[li].at[rs],
                            out_ref.at[pl.ds(own * Mc, Mc), pl.ds(COL0[li], W)],
                            out_sems.at[li],
                        )
                        cp.start()
                        pending[li] = cp
                else:
                    t = g - (N_DEV - 1)
                    idx = (my - DIRS[li] * t) % N_DEV
                    cp = pltpu.make_async_copy(
                        comms[li].at[rs],
                        out_ref.at[pl.ds(idx * Mc, Mc), pl.ds(COL0[li], W)],
                        out_sems.at[li],
                    )
                    cp.start()
                    pending[li] = cp
                if g < NSTEP - 1:
                    pl.semaphore_signal(
                        credits[li], inc=1,
                        device_id=(src[li],),
                        device_id_type=pl.DeviceIdType.MESH,
                    )

        for li in range(N_LANES):
            if pending[li] is not None:
                pending[li].wait()

    scratch = (
        [pltpu.VMEM((2, Mc, W), jnp.bfloat16) for _ in range(N_LANES)]
        + [
            pltpu.SemaphoreType.DMA((N_LANES, 2)),
            pltpu.SemaphoreType.DMA((N_LANES, 2)),
            pltpu.SemaphoreType.DMA((N_LANES,)),
        ]
        + [pltpu.SemaphoreType.REGULAR for _ in range(N_LANES)]
    )

    return pl.pallas_call(
        body,
        out_shape=jax.ShapeDtypeStruct((M, N), jnp.bfloat16),
        in_specs=[
            pl.BlockSpec(memory_space=pltpu.VMEM),
            pl.BlockSpec(memory_space=pltpu.VMEM),
        ],
        out_specs=pl.BlockSpec(memory_space=pl.ANY),
        scratch_shapes=scratch,
        compiler_params=pltpu.CompilerParams(
            collective_id=0,
            vmem_limit_bytes=128 * 1024 * 1024,
        ),
    )(x, w_mat)


# device time: 703391 ns/iter; 2.0058x vs baseline; 1.0389x over previous
import jax
import jax.numpy as jnp
from jax import lax
from jax.experimental import pallas as pl
from jax.experimental.pallas import tpu as pltpu

N_DEV = 8
N_LANES = 4

PERM = (0, 1, 2, 3, 7, 6, 5, 4)


def kernel(x, w_mat):
    x = x.astype(jnp.bfloat16)
    w_mat = w_mat.astype(jnp.bfloat16)

    M, _ = x.shape
    _, N = w_mat.shape
    Mc = M // N_DEV
    W = N // N_LANES
    DIRS = [1 if li % 2 == 0 else -1 for li in range(N_LANES)]
    COL0 = [(li % 2) * (N // 2) + (li // 2) * W for li in range(N_LANES)]
    NSTEP = 2 * (N_DEV - 1)

    def body(x_ref, w_ref, out_ref, *scr):
        comms = scr[:N_LANES]
        send_sems, recv_sems, out_sems = scr[N_LANES : N_LANES + 3]
        credits = scr[N_LANES + 3 :]

        def perm(v):
            return jnp.where(v < 4, v, 11 - v)

        my_mesh = lax.axis_index("i")
        my = perm(my_mesh)
        tgt = [perm((my + d) % N_DEV) for d in DIRS]
        src = [perm((my - d) % N_DEV) for d in DIRS]

        barrier_sem = pltpu.get_barrier_semaphore()
        for nbr in (tgt[0], src[0]):
            pl.semaphore_signal(
                barrier_sem, inc=1,
                device_id=(nbr,), device_id_type=pl.DeviceIdType.MESH,
            )
        pl.semaphore_wait(barrier_sem, 2)

        def partial_chunk(c, li):
            xs = x_ref[pl.ds(c * Mc, Mc), :]
            ws = w_ref[:, COL0[li] : COL0[li] + W]
            return jnp.dot(
                xs, ws, preferred_element_type=jnp.float32
            ).astype(jnp.bfloat16)

        def make_rdma(li, g):
            ss, rs = g % 2, (g + 1) % 2
            return pltpu.make_async_remote_copy(
                src_ref=comms[li].at[ss],
                dst_ref=comms[li].at[rs],
                send_sem=send_sems.at[li, ss],
                recv_sem=recv_sems.at[li, rs],
                device_id=(tgt[li],),
                device_id_type=pl.DeviceIdType.MESH,
            )

        for li in range(N_LANES):
            comms[li][0, :, :] = partial_chunk(my, li)
        rdmas = [None] * N_LANES
        for li in range(N_LANES):
            rdmas[li] = make_rdma(li, 0)
            rdmas[li].start()
        ps = [partial_chunk((my - DIRS[li]) % N_DEV, li) for li in range(N_LANES)]
        pending = [None] * N_LANES

        for g in range(NSTEP):
            rs = (g + 1) % 2
            for li in range(N_LANES):
                rdmas[li].wait()
                if pending[li] is not None:
                    pending[li].wait()
                    pending[li] = None
                if g < N_DEV - 1:
                    acc = comms[li][rs, :, :] + ps[li]
                    if g == N_DEV - 2:
                        acc = jnp.maximum(acc, 0)
                    comms[li][rs, :, :] = acc
                    if g == N_DEV - 2:
                        own = (my + DIRS[li]) % N_DEV
                        cp = pltpu.make_async_copy(
                            comms[li].at[rs],
                            out_ref.at[pl.ds(own * Mc, Mc), pl.ds(COL0[li], W)],
                            out_sems.at[li],
                        )
                        cp.start()
                        pending[li] = cp
                else:
                    t = g - (N_DEV - 1)
                    idx = (my - DIRS[li] * t) % N_DEV
                    cp = pltpu.make_async_copy(
                        comms[li].at[rs],
                        out_ref.at[pl.ds(idx * Mc, Mc), pl.ds(COL0[li], W)],
                        out_sems.at[li],
                    )
                    cp.start()
                    pending[li] = cp
                if g < NSTEP - 1:
                    pl.semaphore_signal(
                        credits[li], inc=1,
                        device_id=(src[li],),
                        device_id_type=pl.DeviceIdType.MESH,
                    )
                if g + 1 < NSTEP:
                    pl.semaphore_wait(credits[li], 1)
                    rdmas[li] = make_rdma(li, g + 1)
                    rdmas[li].start()
                    if g + 1 < N_DEV - 1:
                        ps[li] = partial_chunk(
                            (my - DIRS[li] * (g + 2)) % N_DEV, li
                        )

        for li in range(N_LANES):
            if pending[li] is not None:
                pending[li].wait()

    scratch = (
        [pltpu.VMEM((2, Mc, W), jnp.bfloat16) for _ in range(N_LANES)]
        + [
            pltpu.SemaphoreType.DMA((N_LANES, 2)),
            pltpu.SemaphoreType.DMA((N_LANES, 2)),
            pltpu.SemaphoreType.DMA((N_LANES,)),
        ]
        + [pltpu.SemaphoreType.REGULAR for _ in range(N_LANES)]
    )

    return pl.pallas_call(
        body,
        out_shape=jax.ShapeDtypeStruct((M, N), jnp.bfloat16),
        in_specs=[
            pl.BlockSpec(memory_space=pltpu.VMEM),
            pl.BlockSpec(memory_space=pltpu.VMEM),
        ],
        out_specs=pl.BlockSpec(memory_space=pl.ANY),
        scratch_shapes=scratch,
        compiler_params=pltpu.CompilerParams(
            collective_id=0,
            vmem_limit_bytes=128 * 1024 * 1024,
        ),
    )(x, w_mat)
